# baseline (speedup 1.0000x reference)
import functools

import jax
import jax.numpy as jnp
from jax import lax
from jax.experimental import pallas as pl
from jax.experimental.pallas import tpu as pltpu
from jax.experimental.pallas import tpu_sc as plsc

N = 10000
E = 320000
H = 128
G = 100

NP = 10112
ROWS_PER_TILE = NP // 16
NW = 32
CHUNK = 128
NCHUNK = 80
EPW = CHUNK * NCHUNK
EP = NW * EPW


@functools.lru_cache(maxsize=None)
def _get_sc_hop():
    mesh = plsc.VectorSubcoreMesh(core_axis_name="c", subcore_axis_name="s")
    return functools.partial(
        pl.kernel,
        mesh=mesh,
        out_type=jax.ShapeDtypeStruct((2, NP, H), jnp.float32),
        scratch_types=[
            pltpu.VMEM((NCHUNK // 2, CHUNK), jnp.int32),
            pltpu.VMEM((NCHUNK, CHUNK), jnp.int32),
            pltpu.VMEM((CHUNK, H), jnp.float32),
            pltpu.VMEM((CHUNK, H), jnp.float32),
            pltpu.VMEM_SHARED((NP, H), jnp.float32),
            pltpu.SemaphoreType.DMA,
            pltpu.SemaphoreType.DMA,
        ],
    )(_sc_hop_body)


def _sc_hop(src_p, dst_p, hp, zeros_rows):
    return _get_sc_hop()(src_p, dst_p, hp, zeros_rows)


def _sc_hop_body(src_hbm, dst_hbm, hp_hbm, zeros_hbm, out_hbm,
                 sidx_v, didx_v, rows_a, rows_b, acc_sh, sem_a, sem_b):
    c = lax.axis_index("c")
    s = lax.axis_index("s")
    w = s * 2 + c
    base = s * ROWS_PER_TILE
    NH = NCHUNK // 2
    pltpu.sync_copy(src_hbm.at[w].at[pl.ds(0, NH)], sidx_v)
    pltpu.sync_copy(dst_hbm.at[w], didx_v)
    pltpu.sync_copy(zeros_hbm, acc_sh.at[pl.ds(base, ROWS_PER_TILE)])
    plsc.subcore_barrier()

    pltpu.async_copy(hp_hbm.at[sidx_v.at[0]], rows_a, sem_a)
    pltpu.async_copy(hp_hbm.at[sidx_v.at[1]], rows_b, sem_b)

    def body(g, carry):
        j = 2 * g

        pltpu.make_async_copy(hp_hbm.at[pl.ds(0, CHUNK)], rows_a, sem_a).wait()
        pltpu.sync_copy(rows_a, acc_sh.at[didx_v.at[j]], add=True)
        pltpu.make_async_copy(hp_hbm.at[pl.ds(0, CHUNK)], rows_b, sem_b).wait()
        pltpu.sync_copy(rows_b, acc_sh.at[didx_v.at[j + 1]], add=True)

        @pl.when(j + 2 == NH)
        def _():
            pltpu.sync_copy(src_hbm.at[w].at[pl.ds(NH, NH)], sidx_v)

        @pl.when(g < NCHUNK // 2 - 1)
        def _():
            pltpu.async_copy(hp_hbm.at[sidx_v.at[(j + 2) % NH]], rows_a, sem_a)
            pltpu.async_copy(hp_hbm.at[sidx_v.at[(j + 3) % NH]], rows_b, sem_b)

        return carry

    lax.fori_loop(0, NCHUNK // 2, body, 0)
    plsc.subcore_barrier()
    pltpu.sync_copy(acc_sh.at[pl.ds(base, ROWS_PER_TILE)],
                    out_hbm.at[c].at[pl.ds(base, ROWS_PER_TILE)])


@functools.lru_cache(maxsize=None)
def _get_sc_deg():
    mesh = plsc.VectorSubcoreMesh(core_axis_name="c", subcore_axis_name="s")
    return functools.partial(
        pl.kernel,
        mesh=mesh,
        out_type=jax.ShapeDtypeStruct((2, NP, H), jnp.float32),
        scratch_types=[
            pltpu.VMEM((NCHUNK, CHUNK), jnp.int32),
            pltpu.VMEM((CHUNK, H), jnp.float32),
            pltpu.VMEM_SHARED((NP, H), jnp.float32),
        ],
    )(_sc_deg_body)


def _sc_deg(dst_p, ones_rows, zeros_rows):
    return _get_sc_deg()(dst_p, ones_rows, zeros_rows)


def _sc_deg_body(dst_hbm, ones_hbm, zeros_hbm, out_hbm, didx_v, ones_v, acc_sh):
    c = lax.axis_index("c")
    s = lax.axis_index("s")
    w = s * 2 + c
    base = s * ROWS_PER_TILE
    pltpu.sync_copy(dst_hbm.at[w], didx_v)
    pltpu.sync_copy(zeros_hbm, acc_sh.at[pl.ds(base, ROWS_PER_TILE)])
    pltpu.sync_copy(ones_hbm, ones_v)
    plsc.subcore_barrier()

    def body(j, carry):
        pltpu.sync_copy(ones_v, acc_sh.at[didx_v.at[j]], add=True)
        return carry

    lax.fori_loop(0, NCHUNK, body, 0)
    plsc.subcore_barrier()
    pltpu.sync_copy(acc_sh.at[pl.ds(base, ROWS_PER_TILE)],
                    out_hbm.at[c].at[pl.ds(base, ROWS_PER_TILE)])


RB = NP // 8


def _dis_body(degp_ref, dis_ref):
    deg = degp_ref[0, :, 0:1] + degp_ref[1, :, 0:1] + 1.0
    dis_ref[:, :] = lax.rsqrt(deg)


def _tc_dis(degp):
    return pl.pallas_call(
        _dis_body,
        grid=(8,),
        in_specs=[pl.BlockSpec((2, RB, H), lambda i: (0, i, 0))],
        out_specs=pl.BlockSpec((RB, 1), lambda i: (i, 0)),
        out_shape=jax.ShapeDtypeStruct((NP, 1), jnp.float32),
    )(degp)


def _pro_body(h_ref, W_ref, dis_ref, out_ref, hp_ref):
    h = h_ref[:, :]
    out_ref[:, :] = jnp.dot(h, W_ref[:, :], preferred_element_type=jnp.float32)
    hp_ref[:, :] = dis_ref[:, :] * h


def _tc_prologue(h, Wk, dis):
    return pl.pallas_call(
        _pro_body,
        grid=(8,),
        in_specs=[
            pl.BlockSpec((RB, H), lambda i: (i, 0)),
            pl.BlockSpec((H, H), lambda i: (0, 0)),
            pl.BlockSpec((RB, 1), lambda i: (i, 0)),
        ],
        out_specs=[
            pl.BlockSpec((RB, H), lambda i: (i, 0)),
            pl.BlockSpec((RB, H), lambda i: (i, 0)),
        ],
        out_shape=[
            jax.ShapeDtypeStruct((NP, H), jnp.float32),
            jax.ShapeDtypeStruct((NP, H), jnp.float32),
        ],
    )(h, Wk, dis)


def _hop_tc_body(aggp_ref, hp_ref, dis_ref, W_ref, out_ref, out2_ref, hp2_ref):
    dis = dis_ref[:, :]
    hk = dis * (aggp_ref[0, :, :] + aggp_ref[1, :, :] + hp_ref[:, :])
    out2_ref[:, :] = out_ref[:, :] + jnp.dot(hk, W_ref[:, :], preferred_element_type=jnp.float32)
    hp2_ref[:, :] = dis * hk


def _tc_hop(aggp, hp, dis, Wk, out):
    return pl.pallas_call(
        _hop_tc_body,
        grid=(8,),
        in_specs=[
            pl.BlockSpec((2, RB, H), lambda i: (0, i, 0)),
            pl.BlockSpec((RB, H), lambda i: (i, 0)),
            pl.BlockSpec((RB, 1), lambda i: (i, 0)),
            pl.BlockSpec((H, H), lambda i: (0, 0)),
            pl.BlockSpec((RB, H), lambda i: (i, 0)),
        ],
        out_specs=[
            pl.BlockSpec((RB, H), lambda i: (i, 0)),
            pl.BlockSpec((RB, H), lambda i: (i, 0)),
        ],
        out_shape=[
            jax.ShapeDtypeStruct((NP, H), jnp.float32),
            jax.ShapeDtypeStruct((NP, H), jnp.float32),
        ],
    )(aggp, hp, dis, Wk, out)


def _epi_body(out_ref, b_ref, g_ref, bln_ref, h_ref):
    y = jax.nn.relu(out_ref[:, :] + b_ref[:, :])
    mu = jnp.mean(y, axis=-1, keepdims=True)
    d = y - mu
    var = jnp.mean(d * d, axis=-1, keepdims=True)
    h_ref[:, :] = g_ref[:, :] * d / jnp.sqrt(var + 1e-5) + bln_ref[:, :]


def _tc_epilogue(out, b, g, bln):
    return pl.pallas_call(
        _epi_body,
        grid=(8,),
        in_specs=[
            pl.BlockSpec((RB, H), lambda i: (i, 0)),
            pl.BlockSpec((1, H), lambda i: (0, 0)),
            pl.BlockSpec((1, H), lambda i: (0, 0)),
            pl.BlockSpec((1, H), lambda i: (0, 0)),
        ],
        out_specs=pl.BlockSpec((RB, H), lambda i: (i, 0)),
        out_shape=jax.ShapeDtypeStruct((NP, H), jnp.float32),
    )(out, b, g, bln)


def _head_body(h_ref, bid_ref, gmap_ref, setf_ref, mW_ref, mb_ref,
               f1W_ref, f1b_ref, f2W_ref, f2b_ref, out_ref, t_v, xs_v):
    i = pl.program_id(0)

    @pl.when(i == 0)
    def _():
        cmp = (bid_ref[:, :] < gmap_ref[:, :]).astype(jnp.float32)
        t_v[:, :] = jnp.sum(cmp, axis=1, keepdims=True).astype(jnp.int32) + setf_ref[:, :]
        xs_v[:, :] = jnp.zeros((256, H), jnp.float32)

    rows = lax.broadcasted_iota(jnp.int32, (256, RB), 1) + i * RB
    m = (rows == t_v[:, :]).astype(jnp.float32)
    xs_v[:, :] += jnp.dot(m, h_ref[:, :], preferred_element_type=jnp.float32)

    @pl.when(i == 8 - 1)
    def _():
        xs0 = xs_v[0:128, :]
        xs1 = xs_v[128:256, :]
        x_diff = jnp.abs(xs0 - xs1)
        x_mean = 0.5 * (xs0 + xs1)
        x_max = jnp.maximum(xs0, xs1)
        merged = (
            jnp.dot(x_diff, mW_ref[0:H, :], preferred_element_type=jnp.float32)
            + jnp.dot(x_mean, mW_ref[H:2 * H, :], preferred_element_type=jnp.float32)
            + jnp.dot(x_max, mW_ref[2 * H:3 * H, :], preferred_element_type=jnp.float32)
            + mb_ref[:, :]
        )
        f = jax.nn.relu(jnp.dot(merged, f1W_ref[:, :], preferred_element_type=jnp.float32) + f1b_ref[:, :])
        out_ref[:, :] = jnp.dot(f, f2W_ref[:, :], preferred_element_type=jnp.float32) + f2b_ref[:, :]


def _tc_head(h, bid2d, gmap, setf, merger_W, merger_b, ff1_W, ff1_b, ff2_W, ff2_b):
    return pl.pallas_call(
        _head_body,
        grid=(8,),
        in_specs=[
            pl.BlockSpec((RB, H), lambda i: (i, 0)),
            pl.BlockSpec((1, NP), lambda i: (0, 0)),
            pl.BlockSpec((256, 1), lambda i: (0, 0)),
            pl.BlockSpec((256, 1), lambda i: (0, 0)),
            pl.BlockSpec((3 * H, H), lambda i: (0, 0)),
            pl.BlockSpec((1, H), lambda i: (0, 0)),
            pl.BlockSpec((H, H), lambda i: (0, 0)),
            pl.BlockSpec((1, H), lambda i: (0, 0)),
            pl.BlockSpec((H, H), lambda i: (0, 0)),
            pl.BlockSpec((1, H), lambda i: (0, 0)),
        ],
        out_specs=pl.BlockSpec((128, H), lambda i: (0, 0)),
        out_shape=jax.ShapeDtypeStruct((128, H), jnp.float32),
        scratch_shapes=[
            pltpu.VMEM((256, 1), jnp.int32),
            pltpu.VMEM((256, H), jnp.float32),
        ],
    )(h, bid2d, gmap, setf, merger_W, merger_b, ff1_W, ff1_b, ff2_W, ff2_b)


def kernel(x, edge_index, set_indices, batch_ids, num_graphs, W0, b0, W1, b1,
           ln0_g, ln0_b, ln1_g, ln1_b, merger_W, merger_b, ff1_W, ff1_b, ff2_W, ff2_b):
    src = edge_index[0]
    dst = edge_index[1]

    pad_e = EP - E
    pad_idx = N + (jnp.arange(pad_e, dtype=jnp.int32) % (NP - N))
    src_p = jnp.concatenate([src, pad_idx]).reshape(NW, NCHUNK, CHUNK)
    dst_p = jnp.concatenate([dst, pad_idx]).reshape(NW, NCHUNK, CHUNK)
    x_p = jnp.pad(x, ((0, NP - N), (0, 0)))
    zeros_rows = jnp.zeros((ROWS_PER_TILE, H), jnp.float32)
    ones_rows = jnp.ones((CHUNK, H), jnp.float32)

    degp = _sc_deg(dst_p, ones_rows, zeros_rows)
    dis = _tc_dis(degp)

    def tag(h, W):
        out, hp = _tc_prologue(h, W[0], dis)
        for k in range(1, W.shape[0]):
            aggp = _sc_hop(src_p, dst_p, hp, zeros_rows)
            out, hp = _tc_hop(aggp, hp, dis, W[k], out)
        return out

    h = _tc_epilogue(tag(x_p, W0), b0[None, :], ln0_g[None, :], ln0_b[None, :])
    h = _tc_epilogue(tag(h, W1), b1[None, :], ln1_g[None, :], ln1_b[None, :])

    bid2d = jnp.pad(batch_ids, (0, NP - N), constant_values=G + 1)[None, :]
    gmap = jnp.concatenate([jnp.arange(128, dtype=jnp.int32)] * 2)[:, None]
    setf = jnp.concatenate([
        jnp.pad(set_indices[:, 0], (0, 128 - G), constant_values=-1000000),
        jnp.pad(set_indices[:, 1], (0, 128 - G), constant_values=-1000000),
    ])[:, None]
    out = _tc_head(h, bid2d, gmap, setf, merger_W, merger_b[None, :],
                   ff1_W, ff1_b[None, :], ff2_W, ff2_b[None, :])
    return out[:G]


# speedup vs baseline: 1.2862x; 1.2862x over previous
import functools

import jax
import jax.numpy as jnp
from jax import lax
from jax.experimental import pallas as pl
from jax.experimental.pallas import tpu as pltpu
from jax.experimental.pallas import tpu_sc as plsc

N = 10000
E = 320000
H = 128
G = 100

NP = 10112
ROWS_PER_TILE = NP // 16
NW = 32
CHUNK = 128
NCHUNK = 80
EPW = CHUNK * NCHUNK
EP = NW * EPW


@functools.lru_cache(maxsize=None)
def _get_sc_hop():
    mesh = plsc.VectorSubcoreMesh(core_axis_name="c", subcore_axis_name="s")
    return functools.partial(
        pl.kernel,
        mesh=mesh,
        out_type=jax.ShapeDtypeStruct((2, NP, H), jnp.float32),
        scratch_types=[
            pltpu.VMEM((NCHUNK, CHUNK), jnp.int32),
            pltpu.VMEM((NCHUNK // 2, CHUNK), jnp.int32),
            pltpu.VMEM((CHUNK, H), jnp.float32),
            pltpu.VMEM((CHUNK, H), jnp.float32),
            pltpu.VMEM_SHARED((NP, H), jnp.float32),
            pltpu.SemaphoreType.DMA,
            pltpu.SemaphoreType.DMA,
        ],
    )(_sc_hop_body)


def _sc_hop(src_p, dst_p, hp, zeros_rows):
    return _get_sc_hop()(src_p, dst_p, hp, zeros_rows)


def _sc_hop_body(src_hbm, dst_hbm, hp_hbm, zeros_hbm, out_hbm,
                 sidx_v, didx_v, rows_a, rows_b, acc_sh, sem_a, sem_b):
    c = lax.axis_index("c")
    s = lax.axis_index("s")
    w = s * 2 + c
    base = s * ROWS_PER_TILE
    NH = NCHUNK // 2
    pltpu.sync_copy(src_hbm.at[w], sidx_v)
    pltpu.sync_copy(dst_hbm.at[w].at[pl.ds(0, NH)], didx_v)
    pltpu.sync_copy(zeros_hbm, acc_sh.at[pl.ds(base, ROWS_PER_TILE)])
    plsc.subcore_barrier()

    pltpu.async_copy(hp_hbm.at[sidx_v.at[0]], rows_a, sem_a)
    pltpu.async_copy(hp_hbm.at[sidx_v.at[1]], rows_b, sem_b)

    def body(g, carry):
        j = 2 * g

        pltpu.make_async_copy(hp_hbm.at[pl.ds(0, CHUNK)], rows_a, sem_a).wait()
        pltpu.sync_copy(rows_a, acc_sh.at[didx_v.at[j % NH]], add=True)

        @pl.when(g < NCHUNK // 2 - 1)
        def _():
            pltpu.async_copy(hp_hbm.at[sidx_v.at[j + 2]], rows_a, sem_a)

        pltpu.make_async_copy(hp_hbm.at[pl.ds(0, CHUNK)], rows_b, sem_b).wait()
        pltpu.sync_copy(rows_b, acc_sh.at[didx_v.at[(j + 1) % NH]], add=True)

        @pl.when(j + 2 == NH)
        def _():
            pltpu.sync_copy(dst_hbm.at[w].at[pl.ds(NH, NH)], didx_v)

        @pl.when(g < NCHUNK // 2 - 1)
        def _():
            pltpu.async_copy(hp_hbm.at[sidx_v.at[j + 3]], rows_b, sem_b)

        return carry

    lax.fori_loop(0, NCHUNK // 2, body, 0)
    plsc.subcore_barrier()
    pltpu.sync_copy(acc_sh.at[pl.ds(base, ROWS_PER_TILE)],
                    out_hbm.at[c].at[pl.ds(base, ROWS_PER_TILE)])


@functools.lru_cache(maxsize=None)
def _get_sc_deg():
    mesh = plsc.VectorSubcoreMesh(core_axis_name="c", subcore_axis_name="s")
    return functools.partial(
        pl.kernel,
        mesh=mesh,
        out_type=jax.ShapeDtypeStruct((2, NP, H), jnp.float32),
        scratch_types=[
            pltpu.VMEM((NCHUNK, CHUNK), jnp.int32),
            pltpu.VMEM((CHUNK, H), jnp.float32),
            pltpu.VMEM_SHARED((NP, H), jnp.float32),
        ],
    )(_sc_deg_body)


def _sc_deg(dst_p, ones_rows, zeros_rows):
    return _get_sc_deg()(dst_p, ones_rows, zeros_rows)


def _sc_deg_body(dst_hbm, ones_hbm, zeros_hbm, out_hbm, didx_v, ones_v, acc_sh):
    c = lax.axis_index("c")
    s = lax.axis_index("s")
    w = s * 2 + c
    base = s * ROWS_PER_TILE
    pltpu.sync_copy(dst_hbm.at[w], didx_v)
    pltpu.sync_copy(zeros_hbm, acc_sh.at[pl.ds(base, ROWS_PER_TILE)])
    pltpu.sync_copy(ones_hbm, ones_v)
    plsc.subcore_barrier()

    def body(j, carry):
        pltpu.sync_copy(ones_v, acc_sh.at[didx_v.at[j]], add=True)
        return carry

    lax.fori_loop(0, NCHUNK, body, 0)
    plsc.subcore_barrier()
    pltpu.sync_copy(acc_sh.at[pl.ds(base, ROWS_PER_TILE)],
                    out_hbm.at[c].at[pl.ds(base, ROWS_PER_TILE)])


RB = NP // 8


def _dis_body(degp_ref, dis_ref):
    deg = degp_ref[0, :, 0:1] + degp_ref[1, :, 0:1] + 1.0
    dis_ref[:, :] = 1.0 / jnp.sqrt(deg)


def _tc_dis(degp):
    return pl.pallas_call(
        _dis_body,
        grid=(8,),
        in_specs=[pl.BlockSpec((2, RB, H), lambda i: (0, i, 0))],
        out_specs=pl.BlockSpec((RB, 1), lambda i: (i, 0)),
        out_shape=jax.ShapeDtypeStruct((NP, 1), jnp.float32),
    )(degp)


def _pro_body(h_ref, W_ref, dis_ref, out_ref, hp_ref):
    h = h_ref[:, :]
    out_ref[:, :] = jnp.dot(h, W_ref[:, :], preferred_element_type=jnp.float32)
    hp_ref[:, :] = dis_ref[:, :] * h


def _tc_prologue(h, Wk, dis):
    return pl.pallas_call(
        _pro_body,
        grid=(8,),
        in_specs=[
            pl.BlockSpec((RB, H), lambda i: (i, 0)),
            pl.BlockSpec((H, H), lambda i: (0, 0)),
            pl.BlockSpec((RB, 1), lambda i: (i, 0)),
        ],
        out_specs=[
            pl.BlockSpec((RB, H), lambda i: (i, 0)),
            pl.BlockSpec((RB, H), lambda i: (i, 0)),
        ],
        out_shape=[
            jax.ShapeDtypeStruct((NP, H), jnp.float32),
            jax.ShapeDtypeStruct((NP, H), jnp.float32),
        ],
    )(h, Wk, dis)


def _hop_tc_body(aggp_ref, hp_ref, dis_ref, W_ref, out_ref, out2_ref, hp2_ref):
    dis = dis_ref[:, :]
    hk = dis * (aggp_ref[0, :, :] + aggp_ref[1, :, :] + hp_ref[:, :])
    out2_ref[:, :] = out_ref[:, :] + jnp.dot(hk, W_ref[:, :], preferred_element_type=jnp.float32)
    hp2_ref[:, :] = dis * hk


def _tc_hop(aggp, hp, dis, Wk, out):
    return pl.pallas_call(
        _hop_tc_body,
        grid=(8,),
        in_specs=[
            pl.BlockSpec((2, RB, H), lambda i: (0, i, 0)),
            pl.BlockSpec((RB, H), lambda i: (i, 0)),
            pl.BlockSpec((RB, 1), lambda i: (i, 0)),
            pl.BlockSpec((H, H), lambda i: (0, 0)),
            pl.BlockSpec((RB, H), lambda i: (i, 0)),
        ],
        out_specs=[
            pl.BlockSpec((RB, H), lambda i: (i, 0)),
            pl.BlockSpec((RB, H), lambda i: (i, 0)),
        ],
        out_shape=[
            jax.ShapeDtypeStruct((NP, H), jnp.float32),
            jax.ShapeDtypeStruct((NP, H), jnp.float32),
        ],
    )(aggp, hp, dis, Wk, out)


def _epi_body(out_ref, b_ref, g_ref, bln_ref, h_ref):
    y = jax.nn.relu(out_ref[:, :] + b_ref[:, :])
    mu = jnp.mean(y, axis=-1, keepdims=True)
    d = y - mu
    var = jnp.mean(d * d, axis=-1, keepdims=True)
    h_ref[:, :] = g_ref[:, :] * d / jnp.sqrt(var + 1e-5) + bln_ref[:, :]


def _tc_epilogue(out, b, g, bln):
    return pl.pallas_call(
        _epi_body,
        grid=(8,),
        in_specs=[
            pl.BlockSpec((RB, H), lambda i: (i, 0)),
            pl.BlockSpec((1, H), lambda i: (0, 0)),
            pl.BlockSpec((1, H), lambda i: (0, 0)),
            pl.BlockSpec((1, H), lambda i: (0, 0)),
        ],
        out_specs=pl.BlockSpec((RB, H), lambda i: (i, 0)),
        out_shape=jax.ShapeDtypeStruct((NP, H), jnp.float32),
    )(out, b, g, bln)


def _head_body(h_ref, bid_ref, gmap_ref, setf_ref, mW_ref, mb_ref,
               f1W_ref, f1b_ref, f2W_ref, f2b_ref, out_ref, t_v, xs_v):
    i = pl.program_id(0)

    @pl.when(i == 0)
    def _():
        cmp = (bid_ref[:, :] < gmap_ref[:, :]).astype(jnp.float32)
        t_v[:, :] = jnp.sum(cmp, axis=1, keepdims=True).astype(jnp.int32) + setf_ref[:, :]
        xs_v[:, :] = jnp.zeros((256, H), jnp.float32)

    rows = lax.broadcasted_iota(jnp.int32, (256, RB), 1) + i * RB
    m = (rows == t_v[:, :]).astype(jnp.float32)
    xs_v[:, :] += jnp.dot(m, h_ref[:, :], preferred_element_type=jnp.float32)

    @pl.when(i == 8 - 1)
    def _():
        xs0 = xs_v[0:128, :]
        xs1 = xs_v[128:256, :]
        x_diff = jnp.abs(xs0 - xs1)
        x_mean = 0.5 * (xs0 + xs1)
        x_max = jnp.maximum(xs0, xs1)
        merged = (
            jnp.dot(x_diff, mW_ref[0:H, :], preferred_element_type=jnp.float32)
            + jnp.dot(x_mean, mW_ref[H:2 * H, :], preferred_element_type=jnp.float32)
            + jnp.dot(x_max, mW_ref[2 * H:3 * H, :], preferred_element_type=jnp.float32)
            + mb_ref[:, :]
        )
        f = jax.nn.relu(jnp.dot(merged, f1W_ref[:, :], preferred_element_type=jnp.float32) + f1b_ref[:, :])
        out_ref[:, :] = jnp.dot(f, f2W_ref[:, :], preferred_element_type=jnp.float32) + f2b_ref[:, :]


def _tc_head(h, bid2d, gmap, setf, merger_W, merger_b, ff1_W, ff1_b, ff2_W, ff2_b):
    return pl.pallas_call(
        _head_body,
        grid=(8,),
        in_specs=[
            pl.BlockSpec((RB, H), lambda i: (i, 0)),
            pl.BlockSpec((1, NP), lambda i: (0, 0)),
            pl.BlockSpec((256, 1), lambda i: (0, 0)),
            pl.BlockSpec((256, 1), lambda i: (0, 0)),
            pl.BlockSpec((3 * H, H), lambda i: (0, 0)),
            pl.BlockSpec((1, H), lambda i: (0, 0)),
            pl.BlockSpec((H, H), lambda i: (0, 0)),
            pl.BlockSpec((1, H), lambda i: (0, 0)),
            pl.BlockSpec((H, H), lambda i: (0, 0)),
            pl.BlockSpec((1, H), lambda i: (0, 0)),
        ],
        out_specs=pl.BlockSpec((128, H), lambda i: (0, 0)),
        out_shape=jax.ShapeDtypeStruct((128, H), jnp.float32),
        scratch_shapes=[
            pltpu.VMEM((256, 1), jnp.int32),
            pltpu.VMEM((256, H), jnp.float32),
        ],
    )(h, bid2d, gmap, setf, merger_W, merger_b, ff1_W, ff1_b, ff2_W, ff2_b)


def kernel(x, edge_index, set_indices, batch_ids, num_graphs, W0, b0, W1, b1,
           ln0_g, ln0_b, ln1_g, ln1_b, merger_W, merger_b, ff1_W, ff1_b, ff2_W, ff2_b):
    src = edge_index[0]
    dst = edge_index[1]

    pad_e = EP - E
    pad_idx = N + (jnp.arange(pad_e, dtype=jnp.int32) % (NP - N))
    src_p = jnp.concatenate([src, pad_idx]).reshape(NW, NCHUNK, CHUNK)
    dst_p = jnp.concatenate([dst, pad_idx]).reshape(NW, NCHUNK, CHUNK)
    x_p = jnp.pad(x, ((0, NP - N), (0, 0)))
    zeros_rows = jnp.zeros((ROWS_PER_TILE, H), jnp.float32)
    ones_rows = jnp.ones((CHUNK, H), jnp.float32)

    degp = _sc_deg(dst_p, ones_rows, zeros_rows)
    dis = _tc_dis(degp)

    def tag(h, W):
        out, hp = _tc_prologue(h, W[0], dis)
        for k in range(1, W.shape[0]):
            aggp = _sc_hop(src_p, dst_p, hp, zeros_rows)
            out, hp = _tc_hop(aggp, hp, dis, W[k], out)
        return out

    h = _tc_epilogue(tag(x_p, W0), b0[None, :], ln0_g[None, :], ln0_b[None, :])
    h = _tc_epilogue(tag(h, W1), b1[None, :], ln1_g[None, :], ln1_b[None, :])

    bid2d = jnp.pad(batch_ids, (0, NP - N), constant_values=G + 1)[None, :]
    gmap = jnp.concatenate([jnp.arange(128, dtype=jnp.int32)] * 2)[:, None]
    setf = jnp.concatenate([
        jnp.pad(set_indices[:, 0], (0, 128 - G), constant_values=-1000000),
        jnp.pad(set_indices[:, 1], (0, 128 - G), constant_values=-1000000),
    ])[:, None]
    out = _tc_head(h, bid2d, gmap, setf, merger_W, merger_b[None, :],
                   ff1_W, ff1_b[None, :], ff2_W, ff2_b[None, :])
    return out[:G]


# speedup vs baseline: 1.3179x; 1.0246x over previous
import functools

import jax
import jax.numpy as jnp
from jax import lax
from jax.experimental import pallas as pl
from jax.experimental.pallas import tpu as pltpu
from jax.experimental.pallas import tpu_sc as plsc

N = 10000
E = 320000
H = 128
G = 100

NP = 10112
ROWS_PER_TILE = NP // 16
NW = 32
CHUNK = 128
NCHUNK = 80
EPW = CHUNK * NCHUNK
EP = NW * EPW


@functools.lru_cache(maxsize=None)
def _get_sc_hop():
    mesh = plsc.VectorSubcoreMesh(core_axis_name="c", subcore_axis_name="s")
    return functools.partial(
        pl.kernel,
        mesh=mesh,
        out_type=jax.ShapeDtypeStruct((2, NP, H), jnp.float32),
        scratch_types=[
            pltpu.VMEM((NCHUNK, CHUNK), jnp.int32),
            pltpu.VMEM((NCHUNK // 2, CHUNK), jnp.int32),
            pltpu.VMEM((CHUNK, H), jnp.float32),
            pltpu.VMEM((CHUNK, H), jnp.float32),
            pltpu.VMEM_SHARED((NP, H), jnp.float32),
            pltpu.SemaphoreType.DMA,
            pltpu.SemaphoreType.DMA,
        ],
    )(_sc_hop_body)


def _sc_hop(src_p, dst_p, hp, zeros_rows):
    return _get_sc_hop()(src_p, dst_p, hp, zeros_rows)


def _sc_hop_body(src_hbm, dst_hbm, hp_hbm, zeros_hbm, out_hbm,
                 sidx_v, didx_v, rows_a, rows_b, acc_sh, sem_a, sem_b):
    c = lax.axis_index("c")
    s = lax.axis_index("s")
    w = s * 2 + c
    base = s * ROWS_PER_TILE
    NH = NCHUNK // 2
    pltpu.sync_copy(src_hbm.at[w], sidx_v)
    pltpu.sync_copy(dst_hbm.at[w].at[pl.ds(0, NH)], didx_v)
    pltpu.sync_copy(zeros_hbm, acc_sh.at[pl.ds(base, ROWS_PER_TILE)])
    plsc.subcore_barrier()

    pltpu.async_copy(hp_hbm.at[sidx_v.at[0]], rows_a, sem_a)
    pltpu.async_copy(hp_hbm.at[sidx_v.at[1]], rows_b, sem_b)

    def body(g, carry):
        j = 2 * g

        pltpu.make_async_copy(hp_hbm.at[pl.ds(0, CHUNK)], rows_a, sem_a).wait()
        pltpu.sync_copy(rows_a, acc_sh.at[didx_v.at[j % NH]], add=True)

        @pl.when(g < NCHUNK // 2 - 1)
        def _():
            pltpu.async_copy(hp_hbm.at[sidx_v.at[j + 2]], rows_a, sem_a)

        pltpu.make_async_copy(hp_hbm.at[pl.ds(0, CHUNK)], rows_b, sem_b).wait()
        pltpu.sync_copy(rows_b, acc_sh.at[didx_v.at[(j + 1) % NH]], add=True)

        @pl.when(j + 2 == NH)
        def _():
            pltpu.sync_copy(dst_hbm.at[w].at[pl.ds(NH, NH)], didx_v)

        @pl.when(g < NCHUNK // 2 - 1)
        def _():
            pltpu.async_copy(hp_hbm.at[sidx_v.at[j + 3]], rows_b, sem_b)

        return carry

    lax.fori_loop(0, NCHUNK // 2, body, 0)
    plsc.subcore_barrier()
    pltpu.sync_copy(acc_sh.at[pl.ds(base, ROWS_PER_TILE)],
                    out_hbm.at[c].at[pl.ds(base, ROWS_PER_TILE)])


@functools.lru_cache(maxsize=None)
def _get_sc_deg():
    mesh = plsc.VectorSubcoreMesh(core_axis_name="c", subcore_axis_name="s")
    return functools.partial(
        pl.kernel,
        mesh=mesh,
        out_type=jax.ShapeDtypeStruct((2, NP, H), jnp.float32),
        scratch_types=[
            pltpu.VMEM((NCHUNK, CHUNK), jnp.int32),
            pltpu.VMEM((CHUNK, H), jnp.float32),
            pltpu.VMEM_SHARED((NP, H), jnp.float32),
        ],
    )(_sc_deg_body)


def _sc_deg(dst_p, ones_rows, zeros_rows):
    return _get_sc_deg()(dst_p, ones_rows, zeros_rows)


def _sc_deg_body(dst_hbm, ones_hbm, zeros_hbm, out_hbm, didx_v, ones_v, acc_sh):
    c = lax.axis_index("c")
    s = lax.axis_index("s")
    w = s * 2 + c
    base = s * ROWS_PER_TILE
    pltpu.sync_copy(dst_hbm.at[w], didx_v)
    pltpu.sync_copy(zeros_hbm, acc_sh.at[pl.ds(base, ROWS_PER_TILE)])
    pltpu.sync_copy(ones_hbm, ones_v)
    plsc.subcore_barrier()

    def body(j, carry):
        pltpu.sync_copy(ones_v, acc_sh.at[didx_v.at[j]], add=True)
        return carry

    lax.fori_loop(0, NCHUNK, body, 0)
    plsc.subcore_barrier()
    pltpu.sync_copy(acc_sh.at[pl.ds(base, ROWS_PER_TILE)],
                    out_hbm.at[c].at[pl.ds(base, ROWS_PER_TILE)])


RB = NP // 8


def _pro0_body(h_ref, W_ref, degp_ref, out_ref, hp_ref, dis_ref):
    deg = degp_ref[0, :, 0:1] + degp_ref[1, :, 0:1] + 1.0
    dis = 1.0 / jnp.sqrt(deg)
    dis_ref[:, :] = dis
    h = h_ref[:, :]
    out_ref[:, :] = jnp.dot(h, W_ref[:, :], preferred_element_type=jnp.float32)
    hp_ref[:, :] = dis * h


def _tc_prologue0(h, Wk, degp):
    return pl.pallas_call(
        _pro0_body,
        grid=(8,),
        in_specs=[
            pl.BlockSpec((RB, H), lambda i: (i, 0)),
            pl.BlockSpec((H, H), lambda i: (0, 0)),
            pl.BlockSpec((2, RB, H), lambda i: (0, i, 0)),
        ],
        out_specs=[
            pl.BlockSpec((RB, H), lambda i: (i, 0)),
            pl.BlockSpec((RB, H), lambda i: (i, 0)),
            pl.BlockSpec((RB, 1), lambda i: (i, 0)),
        ],
        out_shape=[
            jax.ShapeDtypeStruct((NP, H), jnp.float32),
            jax.ShapeDtypeStruct((NP, H), jnp.float32),
            jax.ShapeDtypeStruct((NP, 1), jnp.float32),
        ],
    )(h, Wk, degp)


def _epi_pro_body(out_ref, b_ref, g_ref, bln_ref, W_ref, dis_ref,
                  out2_ref, hp_ref):
    y = jax.nn.relu(out_ref[:, :] + b_ref[:, :])
    mu = jnp.mean(y, axis=-1, keepdims=True)
    d = y - mu
    var = jnp.mean(d * d, axis=-1, keepdims=True)
    h = g_ref[:, :] * d / jnp.sqrt(var + 1e-5) + bln_ref[:, :]
    out2_ref[:, :] = jnp.dot(h, W_ref[:, :], preferred_element_type=jnp.float32)
    hp_ref[:, :] = dis_ref[:, :] * h


def _tc_epi_pro(out, b, g, bln, Wk, dis):
    return pl.pallas_call(
        _epi_pro_body,
        grid=(8,),
        in_specs=[
            pl.BlockSpec((RB, H), lambda i: (i, 0)),
            pl.BlockSpec((1, H), lambda i: (0, 0)),
            pl.BlockSpec((1, H), lambda i: (0, 0)),
            pl.BlockSpec((1, H), lambda i: (0, 0)),
            pl.BlockSpec((H, H), lambda i: (0, 0)),
            pl.BlockSpec((RB, 1), lambda i: (i, 0)),
        ],
        out_specs=[
            pl.BlockSpec((RB, H), lambda i: (i, 0)),
            pl.BlockSpec((RB, H), lambda i: (i, 0)),
        ],
        out_shape=[
            jax.ShapeDtypeStruct((NP, H), jnp.float32),
            jax.ShapeDtypeStruct((NP, H), jnp.float32),
        ],
    )(out, b, g, bln, Wk, dis)


def _hop_tc_body(aggp_ref, hp_ref, dis_ref, W_ref, out_ref, out2_ref, hp2_ref):
    dis = dis_ref[:, :]
    hk = dis * (aggp_ref[0, :, :] + aggp_ref[1, :, :] + hp_ref[:, :])
    out2_ref[:, :] = out_ref[:, :] + jnp.dot(hk, W_ref[:, :], preferred_element_type=jnp.float32)
    hp2_ref[:, :] = dis * hk


def _tc_hop(aggp, hp, dis, Wk, out):
    return pl.pallas_call(
        _hop_tc_body,
        grid=(8,),
        in_specs=[
            pl.BlockSpec((2, RB, H), lambda i: (0, i, 0)),
            pl.BlockSpec((RB, H), lambda i: (i, 0)),
            pl.BlockSpec((RB, 1), lambda i: (i, 0)),
            pl.BlockSpec((H, H), lambda i: (0, 0)),
            pl.BlockSpec((RB, H), lambda i: (i, 0)),
        ],
        out_specs=[
            pl.BlockSpec((RB, H), lambda i: (i, 0)),
            pl.BlockSpec((RB, H), lambda i: (i, 0)),
        ],
        out_shape=[
            jax.ShapeDtypeStruct((NP, H), jnp.float32),
            jax.ShapeDtypeStruct((NP, H), jnp.float32),
        ],
    )(aggp, hp, dis, Wk, out)


def _head_body(out_in_ref, b_ref, g_ref, bln_ref, bid_ref, gmap_ref, setf_ref,
               mW_ref, mb_ref, f1W_ref, f1b_ref, f2W_ref, f2b_ref, out_ref,
               t_v, xs_v):
    i = pl.program_id(0)

    @pl.when(i == 0)
    def _():
        cmp = (bid_ref[:, :] < gmap_ref[:, :]).astype(jnp.float32)
        t_v[:, :] = jnp.sum(cmp, axis=1, keepdims=True).astype(jnp.int32) + setf_ref[:, :]
        xs_v[:, :] = jnp.zeros((256, H), jnp.float32)

    y = jax.nn.relu(out_in_ref[:, :] + b_ref[:, :])
    mu = jnp.mean(y, axis=-1, keepdims=True)
    dctr = y - mu
    var = jnp.mean(dctr * dctr, axis=-1, keepdims=True)
    h = g_ref[:, :] * dctr / jnp.sqrt(var + 1e-5) + bln_ref[:, :]

    rows = lax.broadcasted_iota(jnp.int32, (256, RB), 1) + i * RB
    m = (rows == t_v[:, :]).astype(jnp.float32)
    xs_v[:, :] += jnp.dot(m, h, preferred_element_type=jnp.float32)

    @pl.when(i == 8 - 1)
    def _():
        xs0 = xs_v[0:128, :]
        xs1 = xs_v[128:256, :]
        x_diff = jnp.abs(xs0 - xs1)
        x_mean = 0.5 * (xs0 + xs1)
        x_max = jnp.maximum(xs0, xs1)
        merged = (
            jnp.dot(x_diff, mW_ref[0:H, :], preferred_element_type=jnp.float32)
            + jnp.dot(x_mean, mW_ref[H:2 * H, :], preferred_element_type=jnp.float32)
            + jnp.dot(x_max, mW_ref[2 * H:3 * H, :], preferred_element_type=jnp.float32)
            + mb_ref[:, :]
        )
        f = jax.nn.relu(jnp.dot(merged, f1W_ref[:, :], preferred_element_type=jnp.float32) + f1b_ref[:, :])
        out_ref[:, :] = jnp.dot(f, f2W_ref[:, :], preferred_element_type=jnp.float32) + f2b_ref[:, :]


def _tc_head(out_in, b, g, bln, bid2d, gmap, setf,
             merger_W, merger_b, ff1_W, ff1_b, ff2_W, ff2_b):
    return pl.pallas_call(
        _head_body,
        grid=(8,),
        in_specs=[
            pl.BlockSpec((RB, H), lambda i: (i, 0)),
            pl.BlockSpec((1, H), lambda i: (0, 0)),
            pl.BlockSpec((1, H), lambda i: (0, 0)),
            pl.BlockSpec((1, H), lambda i: (0, 0)),
            pl.BlockSpec((1, NP), lambda i: (0, 0)),
            pl.BlockSpec((256, 1), lambda i: (0, 0)),
            pl.BlockSpec((256, 1), lambda i: (0, 0)),
            pl.BlockSpec((3 * H, H), lambda i: (0, 0)),
            pl.BlockSpec((1, H), lambda i: (0, 0)),
            pl.BlockSpec((H, H), lambda i: (0, 0)),
            pl.BlockSpec((1, H), lambda i: (0, 0)),
            pl.BlockSpec((H, H), lambda i: (0, 0)),
            pl.BlockSpec((1, H), lambda i: (0, 0)),
        ],
        out_specs=pl.BlockSpec((128, H), lambda i: (0, 0)),
        out_shape=jax.ShapeDtypeStruct((128, H), jnp.float32),
        scratch_shapes=[
            pltpu.VMEM((256, 1), jnp.int32),
            pltpu.VMEM((256, H), jnp.float32),
        ],
    )(out_in, b, g, bln, bid2d, gmap, setf,
      merger_W, merger_b, ff1_W, ff1_b, ff2_W, ff2_b)


def kernel(x, edge_index, set_indices, batch_ids, num_graphs, W0, b0, W1, b1,
           ln0_g, ln0_b, ln1_g, ln1_b, merger_W, merger_b, ff1_W, ff1_b, ff2_W, ff2_b):
    src = edge_index[0]
    dst = edge_index[1]

    pad_e = EP - E
    pad_idx = N + (jnp.arange(pad_e, dtype=jnp.int32) % (NP - N))
    src_p = jnp.concatenate([src, pad_idx]).reshape(NW, NCHUNK, CHUNK)
    dst_p = jnp.concatenate([dst, pad_idx]).reshape(NW, NCHUNK, CHUNK)
    x_p = jnp.pad(x, ((0, NP - N), (0, 0)))
    zeros_rows = jnp.zeros((ROWS_PER_TILE, H), jnp.float32)
    ones_rows = jnp.ones((CHUNK, H), jnp.float32)

    degp = _sc_deg(dst_p, ones_rows, zeros_rows)

    def hops(out, hp, dis, W):
        for k in range(1, W.shape[0]):
            aggp = _sc_hop(src_p, dst_p, hp, zeros_rows)
            out, hp = _tc_hop(aggp, hp, dis, W[k], out)
        return out

    out0, hp, dis = _tc_prologue0(x_p, W0[0], degp)
    out_l0 = hops(out0, hp, dis, W0)
    out1, hp = _tc_epi_pro(out_l0, b0[None, :], ln0_g[None, :], ln0_b[None, :],
                           W1[0], dis)
    out_l1 = hops(out1, hp, dis, W1)

    bid2d = jnp.pad(batch_ids, (0, NP - N), constant_values=G + 1)[None, :]
    gmap = jnp.concatenate([jnp.arange(128, dtype=jnp.int32)] * 2)[:, None]
    setf = jnp.concatenate([
        jnp.pad(set_indices[:, 0], (0, 128 - G), constant_values=-1000000),
        jnp.pad(set_indices[:, 1], (0, 128 - G), constant_values=-1000000),
    ])[:, None]
    out = _tc_head(out_l1, b1[None, :], ln1_g[None, :], ln1_b[None, :],
                   bid2d, gmap, setf, merger_W, merger_b[None, :],
                   ff1_W, ff1_b[None, :], ff2_W, ff2_b[None, :])
    return out[:G]


# speedup vs baseline: 1.4307x; 1.0856x over previous
import functools

import jax
import jax.numpy as jnp
from jax import lax
from jax.experimental import pallas as pl
from jax.experimental.pallas import tpu as pltpu
from jax.experimental.pallas import tpu_sc as plsc

N = 10000
E = 320000
H = 128
G = 100

NP = 10112
ROWS_PER_TILE = NP // 16
NW = 32
CHUNK = 112
NCHUNK = 90
EPW = CHUNK * NCHUNK
EP = NW * EPW
CHUNK_D = 128
NCHUNK_D = 80
EPW_D = CHUNK_D * NCHUNK_D
EP_D = NW * EPW_D


@functools.lru_cache(maxsize=None)
def _get_sc_hop():
    mesh = plsc.VectorSubcoreMesh(core_axis_name="c", subcore_axis_name="s")
    return functools.partial(
        pl.kernel,
        mesh=mesh,
        out_type=jax.ShapeDtypeStruct((2, NP, H), jnp.float32),
        scratch_types=[
            pltpu.VMEM((2 * CHUNK,), jnp.int32),
            pltpu.VMEM((2 * CHUNK,), jnp.int32),
            pltpu.VMEM((2 * CHUNK,), jnp.int32),
            pltpu.VMEM((2, CHUNK), jnp.int32),
            pltpu.VMEM((2, CHUNK), jnp.int32),
            pltpu.VMEM((2, CHUNK), jnp.int32),
            pltpu.VMEM((CHUNK, H), jnp.float32),
            pltpu.VMEM((CHUNK, H), jnp.float32),
            pltpu.VMEM((CHUNK, H), jnp.float32),
            pltpu.VMEM_SHARED((NP, H), jnp.float32),
            pltpu.SemaphoreType.DMA,
            pltpu.SemaphoreType.DMA,
            pltpu.SemaphoreType.DMA,
            pltpu.SemaphoreType.DMA,
            pltpu.SemaphoreType.DMA,
            pltpu.SemaphoreType.DMA,
            pltpu.SemaphoreType.DMA,
            pltpu.SemaphoreType.DMA,
            pltpu.SemaphoreType.DMA,
        ],
    )(_sc_hop_body)


def _sc_hop(src_p, dst_p, hp, zeros_rows):
    return _get_sc_hop()(src_p, dst_p, hp, zeros_rows)


def _sc_hop_body(src_hbm, dst_hbm, hp_hbm, zeros_hbm, out_hbm,
                 sx0, sx1, sx2, dx0, dx1, dx2, rows0, rows1, rows2, acc_sh,
                 g0, g1, g2, s0, s1, s2, i0, i1, i2):
    c = lax.axis_index("c")
    s = lax.axis_index("s")
    w = s * 2 + c
    base = s * ROWS_PER_TILE
    ebase = w * EPW
    sx = (sx0, sx1, sx2)
    dx = (dx0, dx1, dx2)
    rows = (rows0, rows1, rows2)
    gsem = (g0, g1, g2)
    ssem = (s0, s1, s2)
    isem = (i0, i1, i2)
    pltpu.sync_copy(zeros_hbm, acc_sh.at[pl.ds(base, ROWS_PER_TILE)])
    plsc.subcore_barrier()

    for b in range(3):
        pltpu.sync_copy(src_hbm.at[pl.ds(ebase + b * CHUNK, CHUNK)],
                        sx[b].at[pl.ds(0, CHUNK)])
        pltpu.sync_copy(dst_hbm.at[pl.ds(ebase + b * CHUNK, CHUNK)], dx[b].at[0])
        pltpu.async_copy(hp_hbm.at[sx[b].at[pl.ds(0, CHUNK)]], rows[b], gsem[b])
        off3 = ebase + (b + 3) * CHUNK
        pltpu.async_copy(src_hbm.at[pl.ds(off3, CHUNK)],
                         sx[b].at[pl.ds(CHUNK, CHUNK)], isem[b])
        pltpu.async_copy(dst_hbm.at[pl.ds(off3, CHUNK)], dx[b].at[1], isem[b])

    def body(t, carry):
        p = t % 2
        for b in range(3):
            j = 3 * t + b
            pltpu.make_async_copy(hp_hbm.at[pl.ds(0, CHUNK)], rows[b], gsem[b]).wait()
            pltpu.async_copy(rows[b], acc_sh.at[dx[b].at[p]], ssem[b], add=True)
            pltpu.make_async_copy(hp_hbm.at[pl.ds(0, CHUNK)], rows[b], ssem[b]).wait()

            @pl.when(j + 3 < NCHUNK)
            def _():
                pltpu.make_async_copy(src_hbm.at[pl.ds(0, 2 * CHUNK)], sx[b], isem[b]).wait()
                pltpu.async_copy(hp_hbm.at[sx[b].at[pl.ds((1 - p) * CHUNK, CHUNK)]],
                                 rows[b], gsem[b])

            @pl.when(j + 6 < NCHUNK)
            def _():
                off6 = ebase + (j + 6) * CHUNK
                pltpu.async_copy(src_hbm.at[pl.ds(off6, CHUNK)],
                                 sx[b].at[pl.ds(p * CHUNK, CHUNK)], isem[b])
                pltpu.async_copy(dst_hbm.at[pl.ds(off6, CHUNK)], dx[b].at[p], isem[b])

        return carry

    lax.fori_loop(0, NCHUNK // 3, body, 0)
    plsc.subcore_barrier()
    pltpu.sync_copy(acc_sh.at[pl.ds(base, ROWS_PER_TILE)],
                    out_hbm.at[c].at[pl.ds(base, ROWS_PER_TILE)])


@functools.lru_cache(maxsize=None)
def _get_sc_deg():
    mesh = plsc.VectorSubcoreMesh(core_axis_name="c", subcore_axis_name="s")
    return functools.partial(
        pl.kernel,
        mesh=mesh,
        out_type=jax.ShapeDtypeStruct((2, NP, H), jnp.float32),
        scratch_types=[
            pltpu.VMEM((NCHUNK_D, CHUNK_D), jnp.int32),
            pltpu.VMEM((CHUNK_D, H), jnp.float32),
            pltpu.VMEM_SHARED((NP, H), jnp.float32),
        ],
    )(_sc_deg_body)


def _sc_deg(dst_p, ones_rows, zeros_rows):
    return _get_sc_deg()(dst_p, ones_rows, zeros_rows)


def _sc_deg_body(dst_hbm, ones_hbm, zeros_hbm, out_hbm, didx_v, ones_v, acc_sh):
    c = lax.axis_index("c")
    s = lax.axis_index("s")
    w = s * 2 + c
    base = s * ROWS_PER_TILE
    pltpu.sync_copy(dst_hbm.at[w], didx_v)
    pltpu.sync_copy(zeros_hbm, acc_sh.at[pl.ds(base, ROWS_PER_TILE)])
    pltpu.sync_copy(ones_hbm, ones_v)
    plsc.subcore_barrier()

    def body(j, carry):
        pltpu.sync_copy(ones_v, acc_sh.at[didx_v.at[j]], add=True)
        return carry

    lax.fori_loop(0, NCHUNK_D, body, 0)
    plsc.subcore_barrier()
    pltpu.sync_copy(acc_sh.at[pl.ds(base, ROWS_PER_TILE)],
                    out_hbm.at[c].at[pl.ds(base, ROWS_PER_TILE)])


RB = NP // 8


def _pro0_body(h_ref, W_ref, degp_ref, out_ref, hp_ref, dis_ref):
    deg = degp_ref[0, :, 0:1] + degp_ref[1, :, 0:1] + 1.0
    dis = 1.0 / jnp.sqrt(deg)
    dis_ref[:, :] = dis
    h = h_ref[:, :]
    out_ref[:, :] = jnp.dot(h, W_ref[:, :], preferred_element_type=jnp.float32)
    hp_ref[:, :] = dis * h


def _tc_prologue0(h, Wk, degp):
    return pl.pallas_call(
        _pro0_body,
        grid=(8,),
        in_specs=[
            pl.BlockSpec((RB, H), lambda i: (i, 0)),
            pl.BlockSpec((H, H), lambda i: (0, 0)),
            pl.BlockSpec((2, RB, H), lambda i: (0, i, 0)),
        ],
        out_specs=[
            pl.BlockSpec((RB, H), lambda i: (i, 0)),
            pl.BlockSpec((RB, H), lambda i: (i, 0)),
            pl.BlockSpec((RB, 1), lambda i: (i, 0)),
        ],
        out_shape=[
            jax.ShapeDtypeStruct((NP, H), jnp.float32),
            jax.ShapeDtypeStruct((NP, H), jnp.float32),
            jax.ShapeDtypeStruct((NP, 1), jnp.float32),
        ],
    )(h, Wk, degp)


def _epi_pro_body(out_ref, b_ref, g_ref, bln_ref, W_ref, dis_ref,
                  out2_ref, hp_ref):
    y = jax.nn.relu(out_ref[:, :] + b_ref[:, :])
    mu = jnp.mean(y, axis=-1, keepdims=True)
    d = y - mu
    var = jnp.mean(d * d, axis=-1, keepdims=True)
    h = g_ref[:, :] * d / jnp.sqrt(var + 1e-5) + bln_ref[:, :]
    out2_ref[:, :] = jnp.dot(h, W_ref[:, :], preferred_element_type=jnp.float32)
    hp_ref[:, :] = dis_ref[:, :] * h


def _tc_epi_pro(out, b, g, bln, Wk, dis):
    return pl.pallas_call(
        _epi_pro_body,
        grid=(8,),
        in_specs=[
            pl.BlockSpec((RB, H), lambda i: (i, 0)),
            pl.BlockSpec((1, H), lambda i: (0, 0)),
            pl.BlockSpec((1, H), lambda i: (0, 0)),
            pl.BlockSpec((1, H), lambda i: (0, 0)),
            pl.BlockSpec((H, H), lambda i: (0, 0)),
            pl.BlockSpec((RB, 1), lambda i: (i, 0)),
        ],
        out_specs=[
            pl.BlockSpec((RB, H), lambda i: (i, 0)),
            pl.BlockSpec((RB, H), lambda i: (i, 0)),
        ],
        out_shape=[
            jax.ShapeDtypeStruct((NP, H), jnp.float32),
            jax.ShapeDtypeStruct((NP, H), jnp.float32),
        ],
    )(out, b, g, bln, Wk, dis)


def _hop_tc_body(aggp_ref, hp_ref, dis_ref, W_ref, out_ref, out2_ref, hp2_ref):
    dis = dis_ref[:, :]
    hk = dis * (aggp_ref[0, :, :] + aggp_ref[1, :, :] + hp_ref[:, :])
    out2_ref[:, :] = out_ref[:, :] + jnp.dot(hk, W_ref[:, :], preferred_element_type=jnp.float32)
    hp2_ref[:, :] = dis * hk


def _tc_hop(aggp, hp, dis, Wk, out):
    return pl.pallas_call(
        _hop_tc_body,
        grid=(8,),
        in_specs=[
            pl.BlockSpec((2, RB, H), lambda i: (0, i, 0)),
            pl.BlockSpec((RB, H), lambda i: (i, 0)),
            pl.BlockSpec((RB, 1), lambda i: (i, 0)),
            pl.BlockSpec((H, H), lambda i: (0, 0)),
            pl.BlockSpec((RB, H), lambda i: (i, 0)),
        ],
        out_specs=[
            pl.BlockSpec((RB, H), lambda i: (i, 0)),
            pl.BlockSpec((RB, H), lambda i: (i, 0)),
        ],
        out_shape=[
            jax.ShapeDtypeStruct((NP, H), jnp.float32),
            jax.ShapeDtypeStruct((NP, H), jnp.float32),
        ],
    )(aggp, hp, dis, Wk, out)


def _head_body(out_in_ref, b_ref, g_ref, bln_ref, bid_ref, gmap_ref, setf_ref,
               mW_ref, mb_ref, f1W_ref, f1b_ref, f2W_ref, f2b_ref, out_ref,
               t_v, xs_v):
    i = pl.program_id(0)

    @pl.when(i == 0)
    def _():
        cmp = (bid_ref[:, :] < gmap_ref[:, :]).astype(jnp.float32)
        t_v[:, :] = jnp.sum(cmp, axis=1, keepdims=True).astype(jnp.int32) + setf_ref[:, :]
        xs_v[:, :] = jnp.zeros((256, H), jnp.float32)

    y = jax.nn.relu(out_in_ref[:, :] + b_ref[:, :])
    mu = jnp.mean(y, axis=-1, keepdims=True)
    dctr = y - mu
    var = jnp.mean(dctr * dctr, axis=-1, keepdims=True)
    h = g_ref[:, :] * dctr / jnp.sqrt(var + 1e-5) + bln_ref[:, :]

    rows = lax.broadcasted_iota(jnp.int32, (256, RB), 1) + i * RB
    m = (rows == t_v[:, :]).astype(jnp.float32)
    xs_v[:, :] += jnp.dot(m, h, preferred_element_type=jnp.float32)

    @pl.when(i == 8 - 1)
    def _():
        xs0 = xs_v[0:128, :]
        xs1 = xs_v[128:256, :]
        x_diff = jnp.abs(xs0 - xs1)
        x_mean = 0.5 * (xs0 + xs1)
        x_max = jnp.maximum(xs0, xs1)
        merged = (
            jnp.dot(x_diff, mW_ref[0:H, :], preferred_element_type=jnp.float32)
            + jnp.dot(x_mean, mW_ref[H:2 * H, :], preferred_element_type=jnp.float32)
            + jnp.dot(x_max, mW_ref[2 * H:3 * H, :], preferred_element_type=jnp.float32)
            + mb_ref[:, :]
        )
        f = jax.nn.relu(jnp.dot(merged, f1W_ref[:, :], preferred_element_type=jnp.float32) + f1b_ref[:, :])
        out_ref[:, :] = jnp.dot(f, f2W_ref[:, :], preferred_element_type=jnp.float32) + f2b_ref[:, :]


def _tc_head(out_in, b, g, bln, bid2d, gmap, setf,
             merger_W, merger_b, ff1_W, ff1_b, ff2_W, ff2_b):
    return pl.pallas_call(
        _head_body,
        grid=(8,),
        in_specs=[
            pl.BlockSpec((RB, H), lambda i: (i, 0)),
            pl.BlockSpec((1, H), lambda i: (0, 0)),
            pl.BlockSpec((1, H), lambda i: (0, 0)),
            pl.BlockSpec((1, H), lambda i: (0, 0)),
            pl.BlockSpec((1, NP), lambda i: (0, 0)),
            pl.BlockSpec((256, 1), lambda i: (0, 0)),
            pl.BlockSpec((256, 1), lambda i: (0, 0)),
            pl.BlockSpec((3 * H, H), lambda i: (0, 0)),
            pl.BlockSpec((1, H), lambda i: (0, 0)),
            pl.BlockSpec((H, H), lambda i: (0, 0)),
            pl.BlockSpec((1, H), lambda i: (0, 0)),
            pl.BlockSpec((H, H), lambda i: (0, 0)),
            pl.BlockSpec((1, H), lambda i: (0, 0)),
        ],
        out_specs=pl.BlockSpec((128, H), lambda i: (0, 0)),
        out_shape=jax.ShapeDtypeStruct((128, H), jnp.float32),
        scratch_shapes=[
            pltpu.VMEM((256, 1), jnp.int32),
            pltpu.VMEM((256, H), jnp.float32),
        ],
    )(out_in, b, g, bln, bid2d, gmap, setf,
      merger_W, merger_b, ff1_W, ff1_b, ff2_W, ff2_b)


def kernel(x, edge_index, set_indices, batch_ids, num_graphs, W0, b0, W1, b1,
           ln0_g, ln0_b, ln1_g, ln1_b, merger_W, merger_b, ff1_W, ff1_b, ff2_W, ff2_b):
    src = edge_index[0]
    dst = edge_index[1]

    pad_idx = N + (jnp.arange(EP - E, dtype=jnp.int32) % (NP - N))
    src_p = jnp.concatenate([src, pad_idx])
    dst_p = jnp.concatenate([dst, pad_idx])
    pad_idx_d = N + (jnp.arange(EP_D - E, dtype=jnp.int32) % (NP - N))
    dst_d = jnp.concatenate([dst, pad_idx_d]).reshape(NW, NCHUNK_D, CHUNK_D)
    x_p = jnp.pad(x, ((0, NP - N), (0, 0)))
    zeros_rows = jnp.zeros((ROWS_PER_TILE, H), jnp.float32)
    ones_rows = jnp.ones((CHUNK_D, H), jnp.float32)

    degp = _sc_deg(dst_d, ones_rows, zeros_rows)

    def hops(out, hp, dis, W):
        for k in range(1, W.shape[0]):
            aggp = _sc_hop(src_p, dst_p, hp, zeros_rows)
            out, hp = _tc_hop(aggp, hp, dis, W[k], out)
        return out

    out0, hp, dis = _tc_prologue0(x_p, W0[0], degp)
    out_l0 = hops(out0, hp, dis, W0)
    out1, hp = _tc_epi_pro(out_l0, b0[None, :], ln0_g[None, :], ln0_b[None, :],
                           W1[0], dis)
    out_l1 = hops(out1, hp, dis, W1)

    bid2d = jnp.pad(batch_ids, (0, NP - N), constant_values=G + 1)[None, :]
    gmap = jnp.concatenate([jnp.arange(128, dtype=jnp.int32)] * 2)[:, None]
    setf = jnp.concatenate([
        jnp.pad(set_indices[:, 0], (0, 128 - G), constant_values=-1000000),
        jnp.pad(set_indices[:, 1], (0, 128 - G), constant_values=-1000000),
    ])[:, None]
    out = _tc_head(out_l1, b1[None, :], ln1_g[None, :], ln1_b[None, :],
                   bid2d, gmap, setf, merger_W, merger_b[None, :],
                   ff1_W, ff1_b[None, :], ff2_W, ff2_b[None, :])
    return out[:G]
